# baseline (speedup 1.0000x reference)
import numpy as np
import jax
import jax.numpy as jnp
from jax.experimental import pallas as pl
from jax.experimental.pallas import tpu as pltpu

_RB = 128
_CK = 256
_NCK = 8192 // _CK


def _scan_block(x_ref, u_ref, o_ref):
    u = u_ref[...]
    carry = jnp.zeros((_RB, 1), jnp.float32)
    for k in range(_NCK):
        x = x_ref[:, k * _CK:(k + 1) * _CK]
        excl = jnp.dot(x.astype(jnp.bfloat16), u,
                       preferred_element_type=jnp.float32)
        o_ref[:, k * _CK:(k + 1) * _CK] = excl + carry
        carry = carry + jnp.sum(x, axis=1, keepdims=True)


def kernel(x):
    n_rows, n_cols = x.shape
    u_strict = jnp.asarray(
        np.triu(np.ones((_CK, _CK), dtype=np.float32), k=1),
        dtype=jnp.bfloat16)
    return pl.pallas_call(
        _scan_block,
        grid=(n_rows // _RB,),
        in_specs=[
            pl.BlockSpec((_RB, n_cols), lambda i: (i, 0)),
            pl.BlockSpec((_CK, _CK), lambda i: (0, 0)),
        ],
        out_specs=pl.BlockSpec((_RB, n_cols), lambda i: (i, 0)),
        out_shape=jax.ShapeDtypeStruct(x.shape, x.dtype),
        compiler_params=pltpu.CompilerParams(
            dimension_semantics=("parallel",),
        ),
    )(x, u_strict)


# speedup vs baseline: 1.0029x; 1.0029x over previous
import numpy as np
import jax
import jax.numpy as jnp
from jax.experimental import pallas as pl
from jax.experimental.pallas import tpu as pltpu

_RB = 256
_CK = 512
_NCK = 8192 // _CK


def _scan_block(x_ref, u_ref, o_ref):
    u = u_ref[...]
    carry = jnp.zeros((_RB, 1), jnp.float32)
    for k in range(_NCK):
        x = x_ref[:, k * _CK:(k + 1) * _CK]
        excl = jnp.dot(x.astype(jnp.bfloat16), u,
                       preferred_element_type=jnp.float32)
        o_ref[:, k * _CK:(k + 1) * _CK] = excl + carry
        carry = carry + jnp.sum(x, axis=1, keepdims=True)


def kernel(x):
    n_rows, n_cols = x.shape
    u_strict = jnp.asarray(
        np.triu(np.ones((_CK, _CK), dtype=np.float32), k=1),
        dtype=jnp.bfloat16)
    return pl.pallas_call(
        _scan_block,
        grid=(n_rows // _RB,),
        in_specs=[
            pl.BlockSpec((_RB, n_cols), lambda i: (i, 0)),
            pl.BlockSpec((_CK, _CK), lambda i: (0, 0)),
        ],
        out_specs=pl.BlockSpec((_RB, n_cols), lambda i: (i, 0)),
        out_shape=jax.ShapeDtypeStruct(x.shape, x.dtype),
        compiler_params=pltpu.CompilerParams(
            dimension_semantics=("parallel",),
        ),
    )(x, u_strict)


# speedup vs baseline: 1.0275x; 1.0245x over previous
import numpy as np
import jax
import jax.numpy as jnp
from jax.experimental import pallas as pl
from jax.experimental.pallas import tpu as pltpu

_RB = 256
_CK = 256
_NCK = 8192 // _CK


def _scan_block(x_ref, u_ref, o_ref):
    u = u_ref[...]
    carry = jnp.zeros((_RB, 1), jnp.float32)
    for k in range(_NCK):
        x = x_ref[:, k * _CK:(k + 1) * _CK]
        excl = jnp.dot(x.astype(jnp.bfloat16), u,
                       preferred_element_type=jnp.float32)
        o_ref[:, k * _CK:(k + 1) * _CK] = excl + carry
        carry = carry + jnp.sum(x, axis=1, keepdims=True)


def kernel(x):
    n_rows, n_cols = x.shape
    u_strict = jnp.asarray(
        np.triu(np.ones((_CK, _CK), dtype=np.float32), k=1),
        dtype=jnp.bfloat16)
    return pl.pallas_call(
        _scan_block,
        grid=(n_rows // _RB,),
        in_specs=[
            pl.BlockSpec((_RB, n_cols), lambda i: (i, 0)),
            pl.BlockSpec((_CK, _CK), lambda i: (0, 0)),
        ],
        out_specs=pl.BlockSpec((_RB, n_cols), lambda i: (i, 0)),
        out_shape=jax.ShapeDtypeStruct(x.shape, x.dtype),
        compiler_params=pltpu.CompilerParams(
            dimension_semantics=("parallel",),
        ),
    )(x, u_strict)
